# baseline (speedup 1.0000x reference)
import functools

import jax
import jax.numpy as jnp
import numpy as np
from jax import lax
from jax.experimental import pallas as pl
from jax.experimental.pallas import tpu as pltpu
from jax.experimental.pallas import tpu_sc as plsc


def _threefry2x32(k1, k2, x0, x1):
    ks = (
        np.uint32(k1),
        np.uint32(k2),
        np.uint32(np.uint32(k1) ^ np.uint32(k2) ^ np.uint32(0x1BD11BDA)),
    )
    x = [x0.astype(np.uint32) + ks[0], x1.astype(np.uint32) + ks[1]]

    def rotl(v, r):
        return (v << np.uint32(r)) | (v >> np.uint32(32 - r))

    rotations = ((13, 15, 26, 6), (17, 29, 16, 24))
    for i in range(5):
        for rot in rotations[i % 2]:
            x[0] = x[0] + x[1]
            x[1] = x[0] ^ rotl(x[1], rot)
        x[0] = x[0] + ks[(i + 1) % 3]
        x[1] = x[1] + ks[(i + 2) % 3] + np.uint32(i + 1)
    return x


@functools.lru_cache(maxsize=None)
def _selected_ids(batch_size: int, n_tokens: int) -> tuple[int, ...]:
    zeros2 = np.zeros(2, dtype=np.uint32)
    b1, b2 = _threefry2x32(0, 1, zeros2, np.arange(2, dtype=np.uint32))
    k_hi = (b1[0], b2[0])
    k_lo = (b1[1], b2[1])

    def bits32(k, n):
        c1 = np.zeros(n, dtype=np.uint32)
        c2 = np.arange(n, dtype=np.uint32)
        y1, y2 = _threefry2x32(k[0], k[1], c1, c2)
        return y1 ^ y2

    hi = bits32(k_hi, batch_size)
    lo = bits32(k_lo, batch_size)
    span = np.uint32(n_tokens)
    ones = np.ones((), dtype=np.uint32)
    mult = (ones * 65536) % span
    mult = (mult * mult) % span
    off = ((hi % span) * mult + (lo % span)) % span
    return tuple(int(x) for x in off.astype(np.int32))


def _make_sc_gather(batch_size: int, n_tokens: int, d_model: int, sel):
    mesh = plsc.ScalarSubcoreMesh(axis_name="c", num_cores=1)

    @functools.partial(
        pl.kernel,
        mesh=mesh,
        out_type=jax.ShapeDtypeStruct((batch_size, d_model), jnp.float32),
    )
    def k(tok_hbm, out_hbm):
        for b in range(batch_size):
            pltpu.sync_copy(tok_hbm.at[b, sel[b]], out_hbm.at[b])

    return k


def kernel(tokens):
    batch_size, n_tokens, d_model = tokens.shape
    sel = _selected_ids(batch_size, n_tokens)
    return _make_sc_gather(batch_size, n_tokens, d_model, sel)(tokens)


# speedup vs baseline: 1.1375x; 1.1375x over previous
import functools

import jax
import jax.numpy as jnp
import numpy as np
from jax import lax
from jax.experimental import pallas as pl
from jax.experimental.pallas import tpu as pltpu
from jax.experimental.pallas import tpu_sc as plsc


def _threefry2x32(k1, k2, x0, x1):
    ks = (
        np.uint32(k1),
        np.uint32(k2),
        np.uint32(np.uint32(k1) ^ np.uint32(k2) ^ np.uint32(0x1BD11BDA)),
    )
    x = [x0.astype(np.uint32) + ks[0], x1.astype(np.uint32) + ks[1]]

    def rotl(v, r):
        return (v << np.uint32(r)) | (v >> np.uint32(32 - r))

    rotations = ((13, 15, 26, 6), (17, 29, 16, 24))
    for i in range(5):
        for rot in rotations[i % 2]:
            x[0] = x[0] + x[1]
            x[1] = x[0] ^ rotl(x[1], rot)
        x[0] = x[0] + ks[(i + 1) % 3]
        x[1] = x[1] + ks[(i + 2) % 3] + np.uint32(i + 1)
    return x


@functools.lru_cache(maxsize=None)
def _selected_ids(batch_size: int, n_tokens: int) -> tuple[int, ...]:
    zeros2 = np.zeros(2, dtype=np.uint32)
    b1, b2 = _threefry2x32(0, 1, zeros2, np.arange(2, dtype=np.uint32))
    k_hi = (b1[0], b2[0])
    k_lo = (b1[1], b2[1])

    def bits32(k, n):
        c1 = np.zeros(n, dtype=np.uint32)
        c2 = np.arange(n, dtype=np.uint32)
        y1, y2 = _threefry2x32(k[0], k[1], c1, c2)
        return y1 ^ y2

    hi = bits32(k_hi, batch_size)
    lo = bits32(k_lo, batch_size)
    span = np.uint32(n_tokens)
    ones = np.ones((), dtype=np.uint32)
    mult = (ones * 65536) % span
    mult = (mult * mult) % span
    off = ((hi % span) * mult + (lo % span)) % span
    return tuple(int(x) for x in off.astype(np.int32))


def _make_sc_gather(batch_size: int, n_tokens: int, d_model: int, sel):
    mesh = plsc.VectorSubcoreMesh(
        core_axis_name="c", subcore_axis_name="s", num_cores=1
    )
    n_workers = 16
    splits = max(1, n_workers // batch_size)
    chunk = d_model // splits

    @functools.partial(
        pl.kernel,
        mesh=mesh,
        out_type=jax.ShapeDtypeStruct((batch_size, d_model), jnp.float32),
        scratch_types=[pltpu.VMEM((chunk,), jnp.float32)],
    )
    def k(tok_hbm, out_hbm, seg_v):
        wid = lax.axis_index("s")

        @pl.when(wid < batch_size * splits)
        def _():
            b = wid // splits
            off = (wid % splits) * chunk
            s = jnp.int32(sel[batch_size - 1])
            for i in range(batch_size - 1):
                s = jnp.where(b == i, jnp.int32(sel[i]), s)
            pltpu.sync_copy(tok_hbm.at[b, s, pl.ds(off, chunk)], seg_v)
            pltpu.sync_copy(seg_v, out_hbm.at[b, pl.ds(off, chunk)])

    return k


def kernel(tokens):
    batch_size, n_tokens, d_model = tokens.shape
    sel = _selected_ids(batch_size, n_tokens)
    return _make_sc_gather(batch_size, n_tokens, d_model, sel)(tokens)
